# baseline (speedup 1.0000x reference)
import functools

import jax
import jax.numpy as jnp
from jax import lax
from jax.experimental import pallas as pl
from jax.experimental.pallas import tpu as pltpu
from jax.experimental.pallas import tpu_sc as plsc

_NC = 2
_NS = 16
_NW = _NC * _NS
_L = 16


@functools.cache
def _make_detile(V, D, K):
    tiles_total = V // 8
    tiles_base, tiles_extra = divmod(tiles_total, _NW)
    mesh = plsc.VectorSubcoreMesh(
        core_axis_name="c", subcore_axis_name="s",
        num_cores=_NC, num_subcores=_NS,
    )

    @functools.partial(
        pl.kernel,
        out_type=jax.ShapeDtypeStruct((V * D,), jnp.float32),
        mesh=mesh,
        scratch_types=[
            pltpu.VMEM((2, K, D), jnp.float32),
            pltpu.VMEM((2, K * D), jnp.float32),
            pltpu.SemaphoreType.DMA((2,)),
            pltpu.SemaphoreType.DMA((2,)),
        ],
        compiler_params=pltpu.CompilerParams(use_tc_tiling_on_sc=True),
    )
    def detile_kernel(tbl_hbm, out_hbm, pad_v, flat_v, isem, osem):
        wid = lax.axis_index("s") * _NC + lax.axis_index("c")
        t0 = wid * tiles_base + jnp.minimum(wid, tiles_extra)
        n_tiles = tiles_base + (wid < tiles_extra).astype(jnp.int32)
        row_lo, row_hi = t0 * 8, (t0 + n_tiles) * 8
        n_chunks = (n_tiles * 8 + K - 1) // K

        def chunk_r0(g):
            return jnp.minimum(row_lo + g * K, row_hi - K)

        def in_desc(g, b):
            return pltpu.make_async_copy(
                tbl_hbm.at[pl.ds(chunk_r0(g), K)], pad_v.at[b], isem.at[b])

        def out_desc(g, b):
            return pltpu.make_async_copy(
                flat_v.at[b], out_hbm.at[pl.ds(chunk_r0(g) * D, K * D)],
                osem.at[b])

        for b in range(2):
            @pl.when(b < n_chunks)
            def _():
                in_desc(b, b).start()

        def body(g, carry):
            b = lax.rem(g, 2)

            def run(b):
                in_desc(g, b).wait()

                @pl.when(g >= 2)
                def _():
                    out_desc(g - 2, b).wait()

                @plsc.parallel_loop(0, K, step=1)
                def _(row):
                    for half in range(D // _L):
                        v = pad_v[b, row, pl.ds(half * _L, _L)]
                        flat_v[b, pl.ds(row * D + half * _L, _L)] = v

                out_desc(g, b).start()

                @pl.when(g + 2 < n_chunks)
                def _():
                    in_desc(g + 2, b).start()

            @pl.when(b == 0)
            def _():
                run(0)

            @pl.when(b == 1)
            def _():
                run(1)

            return carry

        lax.fori_loop(0, n_chunks, body, 0)
        for b in range(2):
            @pl.when(n_chunks - 2 + b >= 0)
            def _():
                out_desc(n_chunks - 2 + b, b).wait()

    return detile_kernel


@functools.cache
def _make_gather(H, B0, V, D, N):
    C = B0 // _NW
    n_rounds = H // N
    assert H % N == 0 and B0 % _NW == 0 and C % _L == 0
    mesh = plsc.VectorSubcoreMesh(
        core_axis_name="c", subcore_axis_name="s",
        num_cores=_NC, num_subcores=_NS,
    )

    @functools.partial(
        pl.kernel,
        out_type=jax.ShapeDtypeStruct((H, D, B0), jnp.float32),
        mesh=mesh,
        scratch_types=[
            pltpu.VMEM((H, C), jnp.int32),
            pltpu.VMEM((N, C, D), jnp.float32),
            pltpu.VMEM((N, D, C), jnp.float32),
            pltpu.SemaphoreType.DMA,
            pltpu.SemaphoreType.DMA((N,)),
            pltpu.SemaphoreType.DMA((N,)),
        ],
        compiler_params=pltpu.CompilerParams(
            use_tc_tiling_on_sc=False, needs_layout_passes=False),
    )
    def gather_kernel(xt_hbm, table_hbm, out_hbm, idx_v, rows_v, tr_v,
                      isem, gsem, ssem):
        wid = lax.axis_index("s") * _NC + lax.axis_index("c")
        col0 = wid * C

        def gather_desc(h, b):
            return pltpu.make_async_copy(
                table_hbm.at[idx_v.at[h]], rows_v.at[b], gsem.at[b])

        def store_desc(h, b):
            return pltpu.make_async_copy(
                tr_v.at[b], out_hbm.at[h, :, pl.ds(col0, C)], ssem.at[b])

        pltpu.async_copy(xt_hbm.at[:, pl.ds(col0, C)], idx_v, isem).wait()

        for b in range(N):
            gather_desc(b, b).start()

        lane = jnp.arange(_L, dtype=jnp.int32)

        def round_body(r, carry):
            h0 = r * N
            for b in range(N):
                h = h0 + b
                gather_desc(h, b).wait()
                @pl.when(r > 0)
                def _():
                    store_desc(h - N, b).wait()

                @plsc.parallel_loop(0, C, step=1)
                def _(row):
                    cvec = jnp.zeros((_L,), jnp.int32) + row
                    for half in range(D // _L):
                        v = rows_v[b, row, pl.ds(half * _L, _L)]
                        plsc.store_scatter(
                            tr_v.at[b], [lane + half * _L, cvec], v)

                store_desc(h, b).start()
                @pl.when(h + N < H)
                def _():
                    gather_desc(h + N, b).start()
            return carry

        lax.fori_loop(0, n_rounds, round_body, 0)

        for b in range(N):
            store_desc(H - N + b, b).wait()

    return gather_kernel


def kernel(x, table):
    B0, H = x.shape
    V, D = table.shape
    xt = x.T
    t_lin = _make_detile(V, D, 256)(table)
    out_p = _make_gather(H, B0, V, D, 4)(xt, t_lin.reshape(V, D))
    return out_p.transpose(2, 0, 1)


# speedup vs baseline: 1.1460x; 1.1460x over previous
import functools

import jax
import jax.numpy as jnp
from jax import lax
from jax.experimental import pallas as pl
from jax.experimental.pallas import tpu as pltpu
from jax.experimental.pallas import tpu_sc as plsc

_NC = 2
_NS = 16
_NW = _NC * _NS
_L = 16


@functools.cache
def _make_detile(V, D, K):
    tiles_total = V // 8
    tiles_base, tiles_extra = divmod(tiles_total, _NW)
    mesh = plsc.VectorSubcoreMesh(
        core_axis_name="c", subcore_axis_name="s",
        num_cores=_NC, num_subcores=_NS,
    )

    @functools.partial(
        pl.kernel,
        out_type=jax.ShapeDtypeStruct((V * D,), jnp.float32),
        mesh=mesh,
        scratch_types=[
            pltpu.VMEM((2, K, D), jnp.float32),
            pltpu.VMEM((2, K * D), jnp.float32),
            pltpu.SemaphoreType.DMA((2,)),
            pltpu.SemaphoreType.DMA((2,)),
        ],
        compiler_params=pltpu.CompilerParams(use_tc_tiling_on_sc=True),
    )
    def detile_kernel(tbl_hbm, out_hbm, pad_v, flat_v, isem, osem):
        wid = lax.axis_index("s") * _NC + lax.axis_index("c")
        t0 = wid * tiles_base + jnp.minimum(wid, tiles_extra)
        n_tiles = tiles_base + (wid < tiles_extra).astype(jnp.int32)
        row_lo, row_hi = t0 * 8, (t0 + n_tiles) * 8
        n_chunks = (n_tiles * 8 + K - 1) // K

        def chunk_r0(g):
            return jnp.minimum(row_lo + g * K, row_hi - K)

        def in_desc(g, b):
            return pltpu.make_async_copy(
                tbl_hbm.at[pl.ds(chunk_r0(g), K)], pad_v.at[b], isem.at[b])

        def out_desc(g, b):
            return pltpu.make_async_copy(
                flat_v.at[b], out_hbm.at[pl.ds(chunk_r0(g) * D, K * D)],
                osem.at[b])

        for b in range(2):
            @pl.when(b < n_chunks)
            def _():
                in_desc(b, b).start()

        def body(g, carry):
            b = lax.rem(g, 2)

            def run(b):
                in_desc(g, b).wait()

                @pl.when(g >= 2)
                def _():
                    out_desc(g - 2, b).wait()

                @plsc.parallel_loop(0, K, step=1)
                def _(row):
                    for half in range(D // _L):
                        v = pad_v[b, row, pl.ds(half * _L, _L)]
                        flat_v[b, pl.ds(row * D + half * _L, _L)] = v

                out_desc(g, b).start()

                @pl.when(g + 2 < n_chunks)
                def _():
                    in_desc(g + 2, b).start()

            @pl.when(b == 0)
            def _():
                run(0)

            @pl.when(b == 1)
            def _():
                run(1)

            return carry

        lax.fori_loop(0, n_chunks, body, 0)
        for b in range(2):
            @pl.when(n_chunks - 2 + b >= 0)
            def _():
                out_desc(n_chunks - 2 + b, b).wait()

    return detile_kernel


@functools.cache
def _make_gather(H, B0, V, D, N):
    C = B0 // _NW
    n_rounds = H // N
    assert H % N == 0 and B0 % _NW == 0 and C % _L == 0
    mesh = plsc.VectorSubcoreMesh(
        core_axis_name="c", subcore_axis_name="s",
        num_cores=_NC, num_subcores=_NS,
    )

    DT = D // 8
    @functools.partial(
        pl.kernel,
        out_type=jax.ShapeDtypeStruct((H, DT, (B0 // 128) * 1024),
                                      jnp.float32),
        mesh=mesh,
        scratch_types=[
            pltpu.VMEM((H, C), jnp.int32),
            pltpu.VMEM((N, C, D), jnp.float32),
            pltpu.VMEM((N, DT, 8 * C), jnp.float32),
            pltpu.SemaphoreType.DMA,
            pltpu.SemaphoreType.DMA((N,)),
            pltpu.SemaphoreType.DMA((N,)),
        ],
        compiler_params=pltpu.CompilerParams(
            use_tc_tiling_on_sc=False, needs_layout_passes=False),
    )
    def gather_kernel(xt_hbm, table_hbm, out_hbm, idx_v, rows_v, tr_v,
                      isem, gsem, ssem):
        wid = lax.axis_index("s") * _NC + lax.axis_index("c")
        col0 = wid * C

        def gather_desc(h, b):
            return pltpu.make_async_copy(
                table_hbm.at[idx_v.at[h]], rows_v.at[b], gsem.at[b])

        def store_desc(h, b):
            return pltpu.make_async_copy(
                tr_v.at[b], out_hbm.at[h, :, pl.ds(wid * (8 * C), 8 * C)],
                ssem.at[b])

        pltpu.async_copy(xt_hbm.at[:, pl.ds(col0, C)], idx_v, isem).wait()

        for b in range(N):
            gather_desc(b, b).start()

        lane = jnp.arange(_L, dtype=jnp.int32)

        def round_body(r, carry):
            h0 = r * N
            for b in range(N):
                h = h0 + b
                gather_desc(h, b).wait()
                @pl.when(r > 0)
                def _():
                    store_desc(h - N, b).wait()

                @plsc.parallel_loop(0, C, step=1)
                def _(row):
                    cvec = jnp.zeros((_L,), jnp.int32) + row
                    for half in range(D // _L):
                        d = lane + half * _L
                        v = rows_v[b, row, pl.ds(half * _L, _L)]
                        plsc.store_scatter(
                            tr_v.at[b],
                            [d // 8, (d % 8) * C + cvec], v)

                store_desc(h, b).start()
                @pl.when(h + N < H)
                def _():
                    gather_desc(h + N, b).start()
            return carry

        lax.fori_loop(0, n_rounds, round_body, 0)

        for b in range(N):
            store_desc(H - N + b, b).wait()

    return gather_kernel


def kernel(x, table):
    B0, H = x.shape
    V, D = table.shape
    xt = x.T
    out_p = _make_gather(H, B0, V, D, 4)(xt, table)
    out5 = out_p.reshape(H, D // 8, B0 // 128, 8, 128)
    return out5.transpose(2, 4, 0, 1, 3).reshape(B0, H, D)
